# baseline (speedup 1.0000x reference)
import functools

import jax
import jax.numpy as jnp
from jax import lax
from jax.experimental import pallas as pl
from jax.experimental.pallas import tpu as pltpu
from jax.experimental.pallas import tpu_sc as plsc

N = 10000
NPAD = 10240
E = 160000
EPAD = 163840
CHUNK = 128
N_SC = 2
N_TILE = 16
ROWS_PER_TILE = NPAD // N_TILE
DEG_CHUNKS = EPAD // (N_SC * N_TILE * CHUNK)
AGG_CHUNKS = EPAD // (N_TILE * CHUNK)


def _sc_mesh():
    return plsc.VectorSubcoreMesh(core_axis_name="c", subcore_axis_name="s")


_SC_PARAMS = pltpu.CompilerParams(use_tc_tiling_on_sc=False)
_SC_PARAMS_NL = pltpu.CompilerParams(use_tc_tiling_on_sc=False,
                                     needs_layout_passes=False)


def _deg(dsts, ones_h, zeros_h):
    def body(dst_h, ones_hbm, zeros_hbm, out, ones_v, dst_v, acc):
        cid = lax.axis_index("c")
        sid = lax.axis_index("s")
        pltpu.sync_copy(dst_h.at[cid, sid], dst_v)
        pltpu.sync_copy(ones_hbm, ones_v)
        r0 = sid * ROWS_PER_TILE
        pltpu.sync_copy(zeros_hbm, acc.at[pl.ds(r0, ROWS_PER_TILE)])
        plsc.subcore_barrier()

        def step(j, carry):
            pltpu.sync_copy(ones_v, acc.at[dst_v.at[j]], add=True)
            return carry

        lax.fori_loop(0, DEG_CHUNKS, step, 0)
        plsc.subcore_barrier()
        pltpu.sync_copy(acc.at[pl.ds(r0, ROWS_PER_TILE)],
                        out.at[cid, pl.ds(r0, ROWS_PER_TILE)])

    kfn = pl.kernel(
        body,
        out_type=jax.ShapeDtypeStruct((N_SC, NPAD, 16), jnp.float32),
        mesh=_sc_mesh(),
        scratch_types=[
            pltpu.VMEM((CHUNK, 16), jnp.float32),
            pltpu.VMEM((DEG_CHUNKS, CHUNK), jnp.int32),
            pltpu.VMEM_SHARED((NPAD, 16), jnp.float32),
        ],
        compiler_params=_SC_PARAMS,
    )
    return kfn(dsts, ones_h, zeros_h)


def _agg(g_cat, srcs, dsts, feat):
    n_pass = 2 if feat > 64 else 1
    chunks_per_pass = AGG_CHUNKS // n_pass

    def body(g_h, src_h, dst_h, out, src_v, dst_v, buf0, buf1, acc,
             gs0, gs1, ss0, ss1):
        cid = lax.axis_index("c")
        sid = lax.axis_index("s")
        r0 = sid * ROWS_PER_TILE
        pltpu.sync_copy(g_h.at[pl.ds(cid * NPAD + r0, ROWS_PER_TILE)],
                        acc.at[pl.ds(r0, ROWS_PER_TILE)])
        plsc.subcore_barrier()

        bufs = (buf0, buf1)
        gsems = (gs0, gs1)
        ssems = (ss0, ss1)
        n = chunks_per_pass

        for p in range(n_pass):
            c0 = p * n
            pltpu.sync_copy(src_h.at[cid, sid, pl.ds(c0, n)], src_v)
            pltpu.sync_copy(dst_h.at[sid, pl.ds(c0, n)], dst_v)

            pltpu.async_copy(g_h.at[src_v.at[0]], buf0, gsems[0])

            def step(j, carry):
                for b in range(2):
                    @pl.when(lax.rem(j, 2) == b)
                    def _():
                        bn = (b + 1) % 2
                        @pl.when(j + 1 < n)
                        def _():
                            @pl.when(j >= 1)
                            def _():
                                pltpu.make_async_copy(
                                    bufs[bn], acc.at[dst_v.at[j]],
                                    ssems[bn]).wait()

                            pltpu.async_copy(g_h.at[src_v.at[j + 1]],
                                             bufs[bn], gsems[bn])

                        pltpu.make_async_copy(
                            g_h.at[src_v.at[j]], bufs[b], gsems[b]).wait()
                        pltpu.async_copy(bufs[b], acc.at[dst_v.at[j]],
                                         ssems[b], add=True)

                return carry

            lax.fori_loop(0, n, step, 0)
            for t in range(2):
                j = n - 1 - t
                pltpu.make_async_copy(bufs[j % 2], acc.at[dst_v.at[0]],
                                      ssems[j % 2]).wait()
        plsc.subcore_barrier()
        pltpu.sync_copy(acc.at[pl.ds(r0, ROWS_PER_TILE)],
                        out.at[cid, pl.ds(r0, ROWS_PER_TILE)])

    kfn = pl.kernel(
        body,
        out_type=jax.ShapeDtypeStruct((N_SC, NPAD, feat), jnp.float32),
        mesh=_sc_mesh(),
        scratch_types=[
            pltpu.VMEM((AGG_CHUNKS // n_pass, CHUNK), jnp.int32),
            pltpu.VMEM((AGG_CHUNKS // n_pass, CHUNK), jnp.int32),
            pltpu.VMEM((CHUNK, feat), jnp.float32),
            pltpu.VMEM((CHUNK, feat), jnp.float32),
            pltpu.VMEM_SHARED((NPAD, feat), jnp.float32),
            pltpu.SemaphoreType.DMA,
            pltpu.SemaphoreType.DMA,
            pltpu.SemaphoreType.DMA,
            pltpu.SemaphoreType.DMA,
        ],
        compiler_params=_SC_PARAMS,
    )
    return kfn(g_cat, srcs, dsts)


CHUNK1 = 64
AGG1_CHUNKS = EPAD // (N_TILE * CHUNK1)


def _agg_bf16(g_f32, g_i32, srcs, dsts):
    n_pass = 2
    n = AGG1_CHUNKS // n_pass

    def body(gf_h, gi_h, src_h, dst_h, out, src_v, dst_v, ib0, ib1,
             fb0, fb1, acc, gs0, gs1, ss0, ss1):
        cid = lax.axis_index("c")
        sid = lax.axis_index("s")
        r0 = sid * ROWS_PER_TILE
        pltpu.sync_copy(gf_h.at[pl.ds(cid * NPAD + r0, ROWS_PER_TILE)],
                        acc.at[pl.ds(r0, ROWS_PER_TILE)])
        plsc.subcore_barrier()

        ibufs = (ib0, ib1)
        fbufs = (fb0, fb1)
        gsems = (gs0, gs1)
        ssems = (ss0, ss1)

        def convert(ib, fb):
            def cv(k, c):
                row = k // 4
                gq = lax.rem(k, 4)
                xi = ib[row, pl.ds(gq * 16, 16)]
                ev = plsc.bitcast(lax.shift_left(xi, 16), jnp.float32)
                od = plsc.bitcast(
                    jnp.bitwise_and(xi, jnp.int32(-65536)), jnp.float32)
                fb[row, pl.ds(gq * 32, 16)] = ev
                fb[row, pl.ds(gq * 32 + 16, 16)] = od
                return c

            lax.fori_loop(0, CHUNK1 * 4, cv, 0)

        for p in range(n_pass):
            c0 = p * n
            pltpu.sync_copy(src_h.at[cid, sid, pl.ds(c0, n)], src_v)
            pltpu.sync_copy(dst_h.at[sid, pl.ds(c0, n)], dst_v)

            pltpu.async_copy(gi_h.at[src_v.at[0]], ib0, gsems[0])
            pltpu.async_copy(gi_h.at[src_v.at[1]], ib1, gsems[1])

            def step(j, carry):
                for b in range(2):
                    @pl.when(lax.rem(j, 2) == b)
                    def _():
                        pltpu.make_async_copy(
                            gi_h.at[src_v.at[j]], ibufs[b], gsems[b]).wait()

                        @pl.when(j >= 2)
                        def _():
                            pltpu.make_async_copy(
                                fbufs[b], acc.at[dst_v.at[j]],
                                ssems[b]).wait()

                        convert(ibufs[b], fbufs[b])

                        @pl.when(j + 2 < n)
                        def _():
                            pltpu.async_copy(gi_h.at[src_v.at[j + 2]],
                                             ibufs[b], gsems[b])

                        pltpu.async_copy(fbufs[b], acc.at[dst_v.at[j]],
                                         ssems[b], add=True)

                return carry

            lax.fori_loop(0, n, step, 0)
            for t in range(2):
                j = n - 1 - t
                pltpu.make_async_copy(fbufs[j % 2], acc.at[dst_v.at[0]],
                                      ssems[j % 2]).wait()
        plsc.subcore_barrier()
        pltpu.sync_copy(acc.at[pl.ds(r0, ROWS_PER_TILE)],
                        out.at[cid, pl.ds(r0, ROWS_PER_TILE)])

    kfn = pl.kernel(
        body,
        out_type=jax.ShapeDtypeStruct((N_SC, NPAD, 128), jnp.float32),
        mesh=_sc_mesh(),
        scratch_types=[
            pltpu.VMEM((AGG1_CHUNKS // n_pass, CHUNK1), jnp.int32),
            pltpu.VMEM((AGG1_CHUNKS // n_pass, CHUNK1), jnp.int32),
            pltpu.VMEM((CHUNK1, 64), jnp.int32),
            pltpu.VMEM((CHUNK1, 64), jnp.int32),
            pltpu.VMEM((CHUNK1, 128), jnp.float32),
            pltpu.VMEM((CHUNK1, 128), jnp.float32),
            pltpu.VMEM_SHARED((NPAD, 128), jnp.float32),
            pltpu.SemaphoreType.DMA,
            pltpu.SemaphoreType.DMA,
            pltpu.SemaphoreType.DMA,
            pltpu.SemaphoreType.DMA,
        ],
        compiler_params=_SC_PARAMS_NL,
    )
    return kfn(g_f32, g_i32, srcs, dsts)


def _perm_matrix():
    import numpy as np
    perm = np.zeros((128, 128), dtype=np.float32)
    for c in range(128):
        gq, r = c // 32, c % 32
        m = 32 * gq + 2 * r if r < 16 else 32 * gq + 2 * (r - 16) + 1
        perm[c, m] = 1.0
    return jnp.asarray(perm)


def _dinv_of(d_blk):
    deg = d_blk[0, :, 0:1] + d_blk[1, :, 0:1] + 1.0
    return lax.rsqrt(deg)


def _mm1_scale(x, W1, deg_parts, P):
    bm = 512
    ni = NPAD // bm

    def body(x_ref, w_ref, d_ref, p_ref, g_ref, gb_ref, dv_ref):
        dinv = _dinv_of(d_ref)
        h = jnp.dot(x_ref[...], w_ref[...], preferred_element_type=jnp.float32)
        g = h * dinv
        g_ref[...] = g
        gb_ref[...] = jnp.dot(g, p_ref[...],
                              preferred_element_type=jnp.float32
                              ).astype(jnp.bfloat16)
        dv_ref[...] = jnp.broadcast_to(dinv, (bm, 128))

    return pl.pallas_call(
        body,
        grid=(ni, 2),
        in_specs=[pl.BlockSpec((bm, 256), lambda i, c: (i, 0)),
                  pl.BlockSpec((256, 128), lambda i, c: (0, c)),
                  pl.BlockSpec((2, bm, 16), lambda i, c: (0, i, 0)),
                  pl.BlockSpec((128, 128), lambda i, c: (0, 0))],
        out_specs=[pl.BlockSpec((bm, 128), lambda i, c: (c * ni + i, 0)),
                   pl.BlockSpec((bm, 128), lambda i, c: (c * ni + i, 0)),
                   pl.BlockSpec((bm, 128), lambda i, c: (i, 0))],
        out_shape=[jax.ShapeDtypeStruct((2 * NPAD, 128), jnp.float32),
                   jax.ShapeDtypeStruct((2 * NPAD, 128), jnp.bfloat16),
                   jax.ShapeDtypeStruct((NPAD, 128), jnp.float32)],
    )(x, W1, deg_parts, P)


def _layer2(agg1, dinv_b, b1r, W2):
    bm = 512

    def body(a_ref, dv_ref, b1_ref, w2_ref, g2_ref):
        dinv = dv_ref[:, 0:1]
        z1 = jnp.concatenate([a_ref[0], a_ref[1]], axis=1)
        z1 = jax.nn.relu(z1 * dinv + b1_ref[...])
        h2 = jnp.dot(z1, w2_ref[...], preferred_element_type=jnp.float32)
        g2 = h2 * dinv
        g2_ref[0] = g2[:, :32]
        g2_ref[1] = g2[:, 32:]

    return pl.pallas_call(
        body,
        grid=(NPAD // bm,),
        in_specs=[pl.BlockSpec((2, bm, 128), lambda i: (0, i, 0)),
                  pl.BlockSpec((bm, 128), lambda i: (i, 0)),
                  pl.BlockSpec((1, 256), lambda i: (0, 0)),
                  pl.BlockSpec((256, 64), lambda i: (0, 0))],
        out_specs=pl.BlockSpec((2, bm, 32), lambda i: (0, i, 0)),
        out_shape=jax.ShapeDtypeStruct((2, NPAD, 32), jnp.float32),
    )(agg1, dinv_b, b1r, W2)


def _zout(agg2, dinv_b, b2r):
    bm = 256

    def body(a_ref, dv_ref, b2_ref, z_ref):
        dinv = dv_ref[:, 0:1]
        zc = jnp.concatenate([a_ref[0], a_ref[1]], axis=1)
        z_ref[...] = zc * dinv + b2_ref[...]

    return pl.pallas_call(
        body,
        grid=(NPAD // bm,),
        in_specs=[pl.BlockSpec((2, bm, 32), lambda i: (0, i, 0)),
                  pl.BlockSpec((bm, 128), lambda i: (i, 0)),
                  pl.BlockSpec((1, 64), lambda i: (0, 0))],
        out_specs=pl.BlockSpec((bm, 64), lambda i: (i, 0)),
        out_shape=jax.ShapeDtypeStruct((NPAD, 64), jnp.float32),
    )(agg2, dinv_b, b2r)


def _decoder(z_pad):
    bi = 2048
    bj = 2560

    def body(a_ref, b_ref, o_ref):
        prod = lax.dot_general(a_ref[...], b_ref[...],
                               (((1,), (1,)), ((), ())),
                               preferred_element_type=jnp.float32)
        o_ref[...] = jax.nn.sigmoid(prod)

    return pl.pallas_call(
        body,
        grid=(NPAD // bi, NPAD // bj),
        in_specs=[pl.BlockSpec((bi, 64), lambda i, j: (i, 0)),
                  pl.BlockSpec((bj, 64), lambda i, j: (j, 0))],
        out_specs=pl.BlockSpec((bi, bj), lambda i, j: (i, j)),
        out_shape=jax.ShapeDtypeStruct((N, N), jnp.float32),
    )(z_pad, z_pad)


def kernel(x, edge_index, W1, b1, W2, b2):
    src = edge_index[0].astype(jnp.int32)
    dst = edge_index[1].astype(jnp.int32)
    pad = EPAD - E
    src_p = jnp.concatenate([src, jnp.full((pad,), N, jnp.int32)])
    dst_p = jnp.concatenate([dst, jnp.full((pad,), N, jnp.int32)])

    dst_deg = dst_p.reshape(N_SC, N_TILE, DEG_CHUNKS, CHUNK)
    src_t = src_p.reshape(N_TILE, AGG_CHUNKS, CHUNK)
    srcs_agg = jnp.stack([src_t, src_t + NPAD])
    dst_agg = dst_p.reshape(N_TILE, AGG_CHUNKS, CHUNK)
    ones_h = jnp.ones((CHUNK, 16), jnp.float32)
    zeros_h = jnp.zeros((ROWS_PER_TILE, 16), jnp.float32)

    src_t1 = src_p.reshape(N_TILE, AGG1_CHUNKS, CHUNK1)
    srcs_agg1 = jnp.stack([src_t1, src_t1 + NPAD])
    dst_agg1 = dst_p.reshape(N_TILE, AGG1_CHUNKS, CHUNK1)

    x_pad = jnp.pad(x, ((0, NPAD - N), (0, 0)))
    deg_parts = _deg(dst_deg, ones_h, zeros_h)
    g1_flat, g1_bf, dinv_b = _mm1_scale(x_pad, W1, deg_parts, _perm_matrix())
    g1_i32 = lax.bitcast_convert_type(
        g1_bf.reshape(N_SC * NPAD, 64, 2), jnp.int32)
    agg1 = _agg_bf16(g1_flat, g1_i32, srcs_agg1, dst_agg1)
    g2_cat = _layer2(agg1, dinv_b, b1.reshape(1, 256), W2)
    agg2 = _agg(g2_cat.reshape(N_SC * NPAD, 32), srcs_agg, dst_agg, 32)
    z_pad = _zout(agg2, dinv_b, b2.reshape(1, 64))
    recons = _decoder(z_pad)
    return (recons, z_pad[:N])


# speedup vs baseline: 1.0549x; 1.0549x over previous
import functools

import jax
import jax.numpy as jnp
from jax import lax
from jax.experimental import pallas as pl
from jax.experimental.pallas import tpu as pltpu
from jax.experimental.pallas import tpu_sc as plsc

N = 10000
NPAD = 10240
E = 160000
EPAD = 163840
CHUNK = 128
N_SC = 2
N_TILE = 16
ROWS_PER_TILE = NPAD // N_TILE
DEG_CHUNKS = EPAD // (N_SC * N_TILE * CHUNK)
AGG_CHUNKS = EPAD // (N_TILE * CHUNK)


def _sc_mesh():
    return plsc.VectorSubcoreMesh(core_axis_name="c", subcore_axis_name="s")


_SC_PARAMS = pltpu.CompilerParams(use_tc_tiling_on_sc=False)


def _deg(dsts, ones_h, zeros_h):
    def body(dst_h, ones_hbm, zeros_hbm, out, ones_v, dst_v, acc):
        cid = lax.axis_index("c")
        sid = lax.axis_index("s")
        pltpu.sync_copy(dst_h.at[cid, sid], dst_v)
        pltpu.sync_copy(ones_hbm, ones_v)
        r0 = sid * ROWS_PER_TILE
        pltpu.sync_copy(zeros_hbm, acc.at[pl.ds(r0, ROWS_PER_TILE)])
        plsc.subcore_barrier()

        def step(j, carry):
            pltpu.sync_copy(ones_v, acc.at[dst_v.at[j]], add=True)
            return carry

        lax.fori_loop(0, DEG_CHUNKS, step, 0)
        plsc.subcore_barrier()
        pltpu.sync_copy(acc.at[pl.ds(r0, ROWS_PER_TILE)],
                        out.at[cid, pl.ds(r0, ROWS_PER_TILE)])

    kfn = pl.kernel(
        body,
        out_type=jax.ShapeDtypeStruct((N_SC, NPAD, 16), jnp.float32),
        mesh=_sc_mesh(),
        scratch_types=[
            pltpu.VMEM((CHUNK, 16), jnp.float32),
            pltpu.VMEM((DEG_CHUNKS, CHUNK), jnp.int32),
            pltpu.VMEM_SHARED((NPAD, 16), jnp.float32),
        ],
        compiler_params=_SC_PARAMS,
    )
    return kfn(dsts, ones_h, zeros_h)


def _agg(g_cat, srcs, dsts, feat):
    n_pass = 2 if feat > 64 else 1
    chunks_per_pass = AGG_CHUNKS // n_pass

    def body(g_h, src_h, dst_h, out, src_v, dst_v, buf0, buf1, acc,
             gs0, gs1, ss0, ss1):
        cid = lax.axis_index("c")
        sid = lax.axis_index("s")
        r0 = sid * ROWS_PER_TILE
        pltpu.sync_copy(g_h.at[pl.ds(cid * NPAD + r0, ROWS_PER_TILE)],
                        acc.at[pl.ds(r0, ROWS_PER_TILE)])
        plsc.subcore_barrier()

        bufs = (buf0, buf1)
        gsems = (gs0, gs1)
        ssems = (ss0, ss1)
        n = chunks_per_pass

        for p in range(n_pass):
            c0 = p * n
            pltpu.sync_copy(src_h.at[cid, sid, pl.ds(c0, n)], src_v)
            pltpu.sync_copy(dst_h.at[sid, pl.ds(c0, n)], dst_v)

            pltpu.async_copy(g_h.at[src_v.at[0]], buf0, gsems[0])

            def step(j, carry):
                for b in range(2):
                    @pl.when(lax.rem(j, 2) == b)
                    def _():
                        bn = (b + 1) % 2
                        @pl.when(j + 1 < n)
                        def _():
                            @pl.when(j >= 1)
                            def _():
                                pltpu.make_async_copy(
                                    bufs[bn], acc.at[dst_v.at[j]],
                                    ssems[bn]).wait()

                            pltpu.async_copy(g_h.at[src_v.at[j + 1]],
                                             bufs[bn], gsems[bn])

                        pltpu.make_async_copy(
                            g_h.at[src_v.at[j]], bufs[b], gsems[b]).wait()
                        pltpu.async_copy(bufs[b], acc.at[dst_v.at[j]],
                                         ssems[b], add=True)

                return carry

            lax.fori_loop(0, n, step, 0)
            for t in range(2):
                j = n - 1 - t
                pltpu.make_async_copy(bufs[j % 2], acc.at[dst_v.at[0]],
                                      ssems[j % 2]).wait()
        plsc.subcore_barrier()
        pltpu.sync_copy(acc.at[pl.ds(r0, ROWS_PER_TILE)],
                        out.at[cid, pl.ds(r0, ROWS_PER_TILE)])

    kfn = pl.kernel(
        body,
        out_type=jax.ShapeDtypeStruct((N_SC, NPAD, feat), jnp.float32),
        mesh=_sc_mesh(),
        scratch_types=[
            pltpu.VMEM((AGG_CHUNKS // n_pass, CHUNK), jnp.int32),
            pltpu.VMEM((AGG_CHUNKS // n_pass, CHUNK), jnp.int32),
            pltpu.VMEM((CHUNK, feat), jnp.float32),
            pltpu.VMEM((CHUNK, feat), jnp.float32),
            pltpu.VMEM_SHARED((NPAD, feat), jnp.float32),
            pltpu.SemaphoreType.DMA,
            pltpu.SemaphoreType.DMA,
            pltpu.SemaphoreType.DMA,
            pltpu.SemaphoreType.DMA,
        ],
        compiler_params=_SC_PARAMS,
    )
    return kfn(g_cat, srcs, dsts)


def _dinv_of(d_blk):
    deg = d_blk[0, :, 0:1] + d_blk[1, :, 0:1] + 1.0
    return lax.rsqrt(deg)


def _mm1_scale(x, W1, deg_parts):
    bm = 512
    ni = NPAD // bm

    def body(x_ref, w_ref, d_ref, g_ref, dv_ref):
        dinv = _dinv_of(d_ref)
        h = jnp.dot(x_ref[...], w_ref[...], preferred_element_type=jnp.float32)
        g_ref[...] = h * dinv
        dv_ref[...] = jnp.broadcast_to(dinv, (bm, 128))

    return pl.pallas_call(
        body,
        grid=(ni, 2),
        in_specs=[pl.BlockSpec((bm, 256), lambda i, c: (i, 0)),
                  pl.BlockSpec((256, 128), lambda i, c: (0, c)),
                  pl.BlockSpec((2, bm, 16), lambda i, c: (0, i, 0))],
        out_specs=[pl.BlockSpec((bm, 128), lambda i, c: (c * ni + i, 0)),
                   pl.BlockSpec((bm, 128), lambda i, c: (i, 0))],
        out_shape=[jax.ShapeDtypeStruct((2 * NPAD, 128), jnp.float32),
                   jax.ShapeDtypeStruct((NPAD, 128), jnp.float32)],
    )(x, W1, deg_parts)


def _layer2(agg1, dinv_b, b1r, W2):
    bm = 512

    def body(a_ref, dv_ref, b1_ref, w2_ref, g2_ref):
        dinv = dv_ref[:, 0:1]
        z1 = jnp.concatenate([a_ref[0], a_ref[1]], axis=1)
        z1 = jax.nn.relu(z1 * dinv + b1_ref[...])
        h2 = jnp.dot(z1, w2_ref[...], preferred_element_type=jnp.float32)
        g2 = h2 * dinv
        g2_ref[0] = g2[:, :32]
        g2_ref[1] = g2[:, 32:]

    return pl.pallas_call(
        body,
        grid=(NPAD // bm,),
        in_specs=[pl.BlockSpec((2, bm, 128), lambda i: (0, i, 0)),
                  pl.BlockSpec((bm, 128), lambda i: (i, 0)),
                  pl.BlockSpec((1, 256), lambda i: (0, 0)),
                  pl.BlockSpec((256, 64), lambda i: (0, 0))],
        out_specs=pl.BlockSpec((2, bm, 32), lambda i: (0, i, 0)),
        out_shape=jax.ShapeDtypeStruct((2, NPAD, 32), jnp.float32),
    )(agg1, dinv_b, b1r, W2)


def _zout(agg2, dinv_b, b2r):
    bm = 256

    def body(a_ref, dv_ref, b2_ref, z_ref):
        dinv = dv_ref[:, 0:1]
        zc = jnp.concatenate([a_ref[0], a_ref[1]], axis=1)
        z_ref[...] = zc * dinv + b2_ref[...]

    return pl.pallas_call(
        body,
        grid=(NPAD // bm,),
        in_specs=[pl.BlockSpec((2, bm, 32), lambda i: (0, i, 0)),
                  pl.BlockSpec((bm, 128), lambda i: (i, 0)),
                  pl.BlockSpec((1, 64), lambda i: (0, 0))],
        out_specs=pl.BlockSpec((bm, 64), lambda i: (i, 0)),
        out_shape=jax.ShapeDtypeStruct((NPAD, 64), jnp.float32),
    )(agg2, dinv_b, b2r)


def _decoder(z_pad):
    bi = 2048
    bj = 2560

    def body(a_ref, b_ref, o_ref):
        prod = lax.dot_general(a_ref[...], b_ref[...],
                               (((1,), (1,)), ((), ())),
                               preferred_element_type=jnp.float32)
        o_ref[...] = jax.nn.sigmoid(prod)

    return pl.pallas_call(
        body,
        grid=(NPAD // bi, NPAD // bj),
        in_specs=[pl.BlockSpec((bi, 64), lambda i, j: (i, 0)),
                  pl.BlockSpec((bj, 64), lambda i, j: (j, 0))],
        out_specs=pl.BlockSpec((bi, bj), lambda i, j: (i, j)),
        out_shape=jax.ShapeDtypeStruct((N, N), jnp.float32),
    )(z_pad, z_pad)


def kernel(x, edge_index, W1, b1, W2, b2):
    src = edge_index[0].astype(jnp.int32)
    dst = edge_index[1].astype(jnp.int32)
    pad = EPAD - E
    src_p = jnp.concatenate([src, jnp.full((pad,), N, jnp.int32)])
    dst_p = jnp.concatenate([dst, jnp.full((pad,), N, jnp.int32)])

    dst_deg = dst_p.reshape(N_SC, N_TILE, DEG_CHUNKS, CHUNK)
    src_t = src_p.reshape(N_TILE, AGG_CHUNKS, CHUNK)
    srcs_agg = jnp.stack([src_t, src_t + NPAD])
    dst_agg = dst_p.reshape(N_TILE, AGG_CHUNKS, CHUNK)
    ones_h = jnp.ones((CHUNK, 16), jnp.float32)
    zeros_h = jnp.zeros((ROWS_PER_TILE, 16), jnp.float32)

    x_pad = jnp.pad(x, ((0, NPAD - N), (0, 0)))
    deg_parts = _deg(dst_deg, ones_h, zeros_h)
    g1_flat, dinv_b = _mm1_scale(x_pad, W1, deg_parts)
    agg1 = _agg(g1_flat, srcs_agg, dst_agg, 128)
    g2_cat = _layer2(agg1, dinv_b, b1.reshape(1, 256), W2)
    agg2 = _agg(g2_cat.reshape(N_SC * NPAD, 32), srcs_agg, dst_agg, 32)
    z_pad = _zout(agg2, dinv_b, b2.reshape(1, 64))
    recons = _decoder(z_pad)
    return (recons, z_pad[:N])


# speedup vs baseline: 1.0557x; 1.0007x over previous
import functools

import jax
import jax.numpy as jnp
from jax import lax
from jax.experimental import pallas as pl
from jax.experimental.pallas import tpu as pltpu
from jax.experimental.pallas import tpu_sc as plsc

N = 10000
NPAD = 10240
E = 160000
EPAD = 163840
CHUNK = 128
N_SC = 2
N_TILE = 16
ROWS_PER_TILE = NPAD // N_TILE
DEG_CHUNKS = EPAD // (N_SC * N_TILE * CHUNK)
AGG_CHUNKS = EPAD // (N_TILE * CHUNK)


def _sc_mesh():
    return plsc.VectorSubcoreMesh(core_axis_name="c", subcore_axis_name="s")


_SC_PARAMS = pltpu.CompilerParams(use_tc_tiling_on_sc=False)


def _deg(dsts, ones_h, zeros_h):
    def body(dst_h, ones_hbm, zeros_hbm, out, ones_v, dst_v, acc):
        cid = lax.axis_index("c")
        sid = lax.axis_index("s")
        pltpu.sync_copy(dst_h.at[cid, sid], dst_v)
        pltpu.sync_copy(ones_hbm, ones_v)
        r0 = sid * ROWS_PER_TILE
        pltpu.sync_copy(zeros_hbm, acc.at[pl.ds(r0, ROWS_PER_TILE)])
        plsc.subcore_barrier()

        def step(j, carry):
            pltpu.sync_copy(ones_v, acc.at[dst_v.at[j]], add=True)
            return carry

        lax.fori_loop(0, DEG_CHUNKS, step, 0)
        plsc.subcore_barrier()
        pltpu.sync_copy(acc.at[pl.ds(r0, ROWS_PER_TILE)],
                        out.at[cid, pl.ds(r0, ROWS_PER_TILE)])

    kfn = pl.kernel(
        body,
        out_type=jax.ShapeDtypeStruct((N_SC, NPAD, 16), jnp.float32),
        mesh=_sc_mesh(),
        scratch_types=[
            pltpu.VMEM((CHUNK, 16), jnp.float32),
            pltpu.VMEM((DEG_CHUNKS, CHUNK), jnp.int32),
            pltpu.VMEM_SHARED((NPAD, 16), jnp.float32),
        ],
        compiler_params=_SC_PARAMS,
    )
    return kfn(dsts, ones_h, zeros_h)


def _agg(g_cat, srcs, dsts, feat):
    n_pass = 2 if feat > 64 else 1
    chunks_per_pass = AGG_CHUNKS // n_pass

    def body(g_h, src_h, dst_h, out, src_v, dst_v, buf0, buf1, acc,
             gs0, gs1, ss0, ss1):
        cid = lax.axis_index("c")
        sid = lax.axis_index("s")
        r0 = sid * ROWS_PER_TILE
        pltpu.sync_copy(g_h.at[pl.ds(cid * NPAD + r0, ROWS_PER_TILE)],
                        acc.at[pl.ds(r0, ROWS_PER_TILE)])
        plsc.subcore_barrier()

        bufs = (buf0, buf1)
        gsems = (gs0, gs1)
        ssems = (ss0, ss1)
        n = chunks_per_pass

        for p in range(n_pass):
            c0 = p * n
            pltpu.sync_copy(src_h.at[cid, sid, pl.ds(c0, n)], src_v)
            pltpu.sync_copy(dst_h.at[sid, pl.ds(c0, n)], dst_v)

            pltpu.async_copy(g_h.at[src_v.at[0]], buf0, gsems[0])

            def step(j, carry):
                for b in range(2):
                    @pl.when(lax.rem(j, 2) == b)
                    def _():
                        bn = (b + 1) % 2
                        @pl.when(j + 1 < n)
                        def _():
                            @pl.when(j >= 1)
                            def _():
                                pltpu.make_async_copy(
                                    bufs[bn], acc.at[dst_v.at[j]],
                                    ssems[bn]).wait()

                            pltpu.async_copy(g_h.at[src_v.at[j + 1]],
                                             bufs[bn], gsems[bn])

                        pltpu.make_async_copy(
                            g_h.at[src_v.at[j]], bufs[b], gsems[b]).wait()
                        pltpu.async_copy(bufs[b], acc.at[dst_v.at[j]],
                                         ssems[b], add=True)

                return carry

            lax.fori_loop(0, n, step, 0)
            for t in range(2):
                j = n - 1 - t
                pltpu.make_async_copy(bufs[j % 2], acc.at[dst_v.at[0]],
                                      ssems[j % 2]).wait()
        plsc.subcore_barrier()
        pltpu.sync_copy(acc.at[pl.ds(r0, ROWS_PER_TILE)],
                        out.at[cid, pl.ds(r0, ROWS_PER_TILE)])

    kfn = pl.kernel(
        body,
        out_type=jax.ShapeDtypeStruct((N_SC, NPAD, feat), jnp.float32),
        mesh=_sc_mesh(),
        scratch_types=[
            pltpu.VMEM((AGG_CHUNKS // n_pass, CHUNK), jnp.int32),
            pltpu.VMEM((AGG_CHUNKS // n_pass, CHUNK), jnp.int32),
            pltpu.VMEM((CHUNK, feat), jnp.float32),
            pltpu.VMEM((CHUNK, feat), jnp.float32),
            pltpu.VMEM_SHARED((NPAD, feat), jnp.float32),
            pltpu.SemaphoreType.DMA,
            pltpu.SemaphoreType.DMA,
            pltpu.SemaphoreType.DMA,
            pltpu.SemaphoreType.DMA,
        ],
        compiler_params=_SC_PARAMS,
    )
    return kfn(g_cat, srcs, dsts)


def _dinv_of(d_blk):
    deg = d_blk[0, :, 0:1] + d_blk[1, :, 0:1] + 1.0
    return lax.rsqrt(deg)


def _mm1_scale(x, W1, deg_parts):
    bm = 512
    ni = NPAD // bm

    def body(x_ref, w_ref, d_ref, g_ref, dv_ref):
        dinv = _dinv_of(d_ref)
        h = jnp.dot(x_ref[...], w_ref[...], preferred_element_type=jnp.float32)
        g_ref[...] = h * dinv
        dv_ref[...] = jnp.broadcast_to(dinv, (bm, 128))

    return pl.pallas_call(
        body,
        grid=(ni, 2),
        in_specs=[pl.BlockSpec((bm, 256), lambda i, c: (i, 0)),
                  pl.BlockSpec((256, 128), lambda i, c: (0, c)),
                  pl.BlockSpec((2, bm, 16), lambda i, c: (0, i, 0))],
        out_specs=[pl.BlockSpec((bm, 128), lambda i, c: (c * ni + i, 0)),
                   pl.BlockSpec((bm, 128), lambda i, c: (i, 0))],
        out_shape=[jax.ShapeDtypeStruct((2 * NPAD, 128), jnp.float32),
                   jax.ShapeDtypeStruct((NPAD, 128), jnp.float32)],
    )(x, W1, deg_parts)


def _layer2(agg1, dinv_b, b1r, W2):
    bm = 512

    def body(a_ref, dv_ref, b1_ref, w2_ref, g2_ref):
        dinv = dv_ref[:, 0:1]
        z1 = jnp.concatenate([a_ref[0], a_ref[1]], axis=1)
        z1 = jax.nn.relu(z1 * dinv + b1_ref[...])
        h2 = jnp.dot(z1, w2_ref[...], preferred_element_type=jnp.float32)
        g2 = h2 * dinv
        g2_ref[0] = g2[:, :32]
        g2_ref[1] = g2[:, 32:]

    return pl.pallas_call(
        body,
        grid=(NPAD // bm,),
        in_specs=[pl.BlockSpec((2, bm, 128), lambda i: (0, i, 0)),
                  pl.BlockSpec((bm, 128), lambda i: (i, 0)),
                  pl.BlockSpec((1, 256), lambda i: (0, 0)),
                  pl.BlockSpec((256, 64), lambda i: (0, 0))],
        out_specs=pl.BlockSpec((2, bm, 32), lambda i: (0, i, 0)),
        out_shape=jax.ShapeDtypeStruct((2, NPAD, 32), jnp.float32),
    )(agg1, dinv_b, b1r, W2)


def _zout(agg2, dinv_b, b2r):
    bm = 256

    def body(a_ref, dv_ref, b2_ref, z_ref):
        dinv = dv_ref[:, 0:1]
        zc = jnp.concatenate([a_ref[0], a_ref[1]], axis=1)
        z_ref[...] = zc * dinv + b2_ref[...]

    return pl.pallas_call(
        body,
        grid=(NPAD // bm,),
        in_specs=[pl.BlockSpec((2, bm, 32), lambda i: (0, i, 0)),
                  pl.BlockSpec((bm, 128), lambda i: (i, 0)),
                  pl.BlockSpec((1, 64), lambda i: (0, 0))],
        out_specs=pl.BlockSpec((bm, 64), lambda i: (i, 0)),
        out_shape=jax.ShapeDtypeStruct((NPAD, 64), jnp.float32),
    )(agg2, dinv_b, b2r)


def _decoder(z_pad):
    bi = 2048
    bj = 2048

    def body(a_ref, b_ref, o_ref):
        prod = lax.dot_general(a_ref[...], b_ref[...],
                               (((1,), (1,)), ((), ())),
                               preferred_element_type=jnp.float32)
        o_ref[...] = jax.nn.sigmoid(prod)

    return pl.pallas_call(
        body,
        grid=(NPAD // bi, NPAD // bj),
        in_specs=[pl.BlockSpec((bi, 64), lambda i, j: (i, 0)),
                  pl.BlockSpec((bj, 64), lambda i, j: (j, 0))],
        out_specs=pl.BlockSpec((bi, bj), lambda i, j: (i, j)),
        out_shape=jax.ShapeDtypeStruct((N, N), jnp.float32),
    )(z_pad, z_pad)


def kernel(x, edge_index, W1, b1, W2, b2):
    src = edge_index[0].astype(jnp.int32)
    dst = edge_index[1].astype(jnp.int32)
    pad = EPAD - E
    src_p = jnp.concatenate([src, jnp.full((pad,), N, jnp.int32)])
    dst_p = jnp.concatenate([dst, jnp.full((pad,), N, jnp.int32)])

    dst_deg = dst_p.reshape(N_SC, N_TILE, DEG_CHUNKS, CHUNK)
    src_t = src_p.reshape(N_TILE, AGG_CHUNKS, CHUNK)
    srcs_agg = jnp.stack([src_t, src_t + NPAD])
    dst_agg = dst_p.reshape(N_TILE, AGG_CHUNKS, CHUNK)
    ones_h = jnp.ones((CHUNK, 16), jnp.float32)
    zeros_h = jnp.zeros((ROWS_PER_TILE, 16), jnp.float32)

    x_pad = jnp.pad(x, ((0, NPAD - N), (0, 0)))
    deg_parts = _deg(dst_deg, ones_h, zeros_h)
    g1_flat, dinv_b = _mm1_scale(x_pad, W1, deg_parts)
    agg1 = _agg(g1_flat, srcs_agg, dst_agg, 128)
    g2_cat = _layer2(agg1, dinv_b, b1.reshape(1, 256), W2)
    agg2 = _agg(g2_cat.reshape(N_SC * NPAD, 32), srcs_agg, dst_agg, 32)
    z_pad = _zout(agg2, dinv_b, b2.reshape(1, 64))
    recons = _decoder(z_pad)
    return (recons, z_pad[:N])


# speedup vs baseline: 1.0565x; 1.0008x over previous
import jax
import jax.numpy as jnp
from jax import lax
from jax.experimental import pallas as pl
from jax.experimental.pallas import tpu as pltpu
from jax.experimental.pallas import tpu_sc as plsc

N = 10000
NPAD = 10240
E = 160000
EPAD = 163840
CHUNK = 128
N_SC = 2
N_TILE = 16
ROWS_PER_TILE = NPAD // N_TILE
DEG_CHUNKS = EPAD // (N_SC * N_TILE * CHUNK)
AGG_CHUNKS = EPAD // (N_TILE * CHUNK)


def _sc_mesh():
    return plsc.VectorSubcoreMesh(core_axis_name="c", subcore_axis_name="s")


_SC_PARAMS = pltpu.CompilerParams(use_tc_tiling_on_sc=False)


def _deg(dsts, ones_h, zeros_h):
    def body(dst_h, ones_hbm, zeros_hbm, out, ones_v, dst_v, acc):
        cid = lax.axis_index("c")
        sid = lax.axis_index("s")
        pltpu.sync_copy(dst_h.at[cid, sid], dst_v)
        pltpu.sync_copy(ones_hbm, ones_v)
        r0 = sid * ROWS_PER_TILE
        pltpu.sync_copy(zeros_hbm, acc.at[pl.ds(r0, ROWS_PER_TILE)])
        plsc.subcore_barrier()

        def step(j, carry):
            pltpu.sync_copy(ones_v, acc.at[dst_v.at[j]], add=True)
            return carry

        lax.fori_loop(0, DEG_CHUNKS, step, 0)
        plsc.subcore_barrier()
        pltpu.sync_copy(acc.at[pl.ds(r0, ROWS_PER_TILE)],
                        out.at[cid, pl.ds(r0, ROWS_PER_TILE)])

    kfn = pl.kernel(
        body,
        out_type=jax.ShapeDtypeStruct((N_SC, NPAD, 16), jnp.float32),
        mesh=_sc_mesh(),
        scratch_types=[
            pltpu.VMEM((CHUNK, 16), jnp.float32),
            pltpu.VMEM((DEG_CHUNKS, CHUNK), jnp.int32),
            pltpu.VMEM_SHARED((NPAD, 16), jnp.float32),
        ],
        compiler_params=_SC_PARAMS,
    )
    return kfn(dsts, ones_h, zeros_h)


def _agg(g_cat, srcs, dsts, feat):
    n_pass = 2 if feat > 64 else 1
    chunks_per_pass = AGG_CHUNKS // n_pass

    def body(g_h, src_h, dst_h, out, src_v, dst_v, buf0, buf1, acc,
             gs0, gs1, ss0, ss1):
        cid = lax.axis_index("c")
        sid = lax.axis_index("s")
        r0 = sid * ROWS_PER_TILE
        pltpu.sync_copy(g_h.at[pl.ds(cid * NPAD + r0, ROWS_PER_TILE)],
                        acc.at[pl.ds(r0, ROWS_PER_TILE)])
        plsc.subcore_barrier()

        bufs = (buf0, buf1)
        gsems = (gs0, gs1)
        ssems = (ss0, ss1)
        n = chunks_per_pass

        for p in range(n_pass):
            c0 = p * n
            pltpu.sync_copy(src_h.at[cid, sid, pl.ds(c0, n)], src_v)
            pltpu.sync_copy(dst_h.at[sid, pl.ds(c0, n)], dst_v)

            pltpu.async_copy(g_h.at[src_v.at[0]], buf0, gsems[0])

            def step(j, carry):
                for b in range(2):
                    @pl.when(lax.rem(j, 2) == b)
                    def _():
                        bn = (b + 1) % 2
                        @pl.when(j + 1 < n)
                        def _():
                            @pl.when(j >= 1)
                            def _():
                                pltpu.make_async_copy(
                                    bufs[bn], acc.at[dst_v.at[j]],
                                    ssems[bn]).wait()

                            pltpu.async_copy(g_h.at[src_v.at[j + 1]],
                                             bufs[bn], gsems[bn])

                        pltpu.make_async_copy(
                            g_h.at[src_v.at[j]], bufs[b], gsems[b]).wait()
                        pltpu.async_copy(bufs[b], acc.at[dst_v.at[j]],
                                         ssems[b], add=True)

                return carry

            lax.fori_loop(0, n, step, 0)
            for t in range(2):
                j = n - 1 - t
                pltpu.make_async_copy(bufs[j % 2], acc.at[dst_v.at[0]],
                                      ssems[j % 2]).wait()
        plsc.subcore_barrier()
        pltpu.sync_copy(acc.at[pl.ds(r0, ROWS_PER_TILE)],
                        out.at[cid, pl.ds(r0, ROWS_PER_TILE)])

    kfn = pl.kernel(
        body,
        out_type=jax.ShapeDtypeStruct((N_SC, NPAD, feat), jnp.float32),
        mesh=_sc_mesh(),
        scratch_types=[
            pltpu.VMEM((AGG_CHUNKS // n_pass, CHUNK), jnp.int32),
            pltpu.VMEM((AGG_CHUNKS // n_pass, CHUNK), jnp.int32),
            pltpu.VMEM((CHUNK, feat), jnp.float32),
            pltpu.VMEM((CHUNK, feat), jnp.float32),
            pltpu.VMEM_SHARED((NPAD, feat), jnp.float32),
            pltpu.SemaphoreType.DMA,
            pltpu.SemaphoreType.DMA,
            pltpu.SemaphoreType.DMA,
            pltpu.SemaphoreType.DMA,
        ],
        compiler_params=_SC_PARAMS,
    )
    return kfn(g_cat, srcs, dsts)


def _dinv_of(d_blk):
    deg = d_blk[0, :, 0:1] + d_blk[1, :, 0:1] + 1.0
    return lax.rsqrt(deg)


def _mm1_scale(x, W1, deg_parts):
    bm = 512
    ni = NPAD // bm

    def body(x_ref, w_ref, d_ref, g_ref, dv_ref):
        dinv = _dinv_of(d_ref)
        h = jnp.dot(x_ref[...], w_ref[...], preferred_element_type=jnp.float32)
        g_ref[...] = h * dinv
        dv_ref[...] = jnp.broadcast_to(dinv, (bm, 128))

    return pl.pallas_call(
        body,
        grid=(ni, 2),
        in_specs=[pl.BlockSpec((bm, 256), lambda i, c: (i, 0)),
                  pl.BlockSpec((256, 128), lambda i, c: (0, c)),
                  pl.BlockSpec((2, bm, 16), lambda i, c: (0, i, 0))],
        out_specs=[pl.BlockSpec((bm, 128), lambda i, c: (c * ni + i, 0)),
                   pl.BlockSpec((bm, 128), lambda i, c: (i, 0))],
        out_shape=[jax.ShapeDtypeStruct((2 * NPAD, 128), jnp.float32),
                   jax.ShapeDtypeStruct((NPAD, 128), jnp.float32)],
    )(x, W1, deg_parts)


def _layer2(agg1, dinv_b, b1r, W2):
    bm = 512

    def body(a_ref, dv_ref, b1_ref, w2_ref, g2_ref):
        dinv = dv_ref[:, 0:1]
        z1 = jnp.concatenate([a_ref[0], a_ref[1]], axis=1)
        z1 = jax.nn.relu(z1 * dinv + b1_ref[...])
        h2 = jnp.dot(z1, w2_ref[...], preferred_element_type=jnp.float32)
        g2 = h2 * dinv
        g2_ref[0] = g2[:, :32]
        g2_ref[1] = g2[:, 32:]

    return pl.pallas_call(
        body,
        grid=(NPAD // bm,),
        in_specs=[pl.BlockSpec((2, bm, 128), lambda i: (0, i, 0)),
                  pl.BlockSpec((bm, 128), lambda i: (i, 0)),
                  pl.BlockSpec((1, 256), lambda i: (0, 0)),
                  pl.BlockSpec((256, 64), lambda i: (0, 0))],
        out_specs=pl.BlockSpec((2, bm, 32), lambda i: (0, i, 0)),
        out_shape=jax.ShapeDtypeStruct((2, NPAD, 32), jnp.float32),
    )(agg1, dinv_b, b1r, W2)


def _zout(agg2, dinv_b, b2r):
    bm = 256

    def body(a_ref, dv_ref, b2_ref, z_ref):
        dinv = dv_ref[:, 0:1]
        zc = jnp.concatenate([a_ref[0], a_ref[1]], axis=1)
        z_ref[...] = zc * dinv + b2_ref[...]

    return pl.pallas_call(
        body,
        grid=(NPAD // bm,),
        in_specs=[pl.BlockSpec((2, bm, 32), lambda i: (0, i, 0)),
                  pl.BlockSpec((bm, 128), lambda i: (i, 0)),
                  pl.BlockSpec((1, 64), lambda i: (0, 0))],
        out_specs=pl.BlockSpec((bm, 64), lambda i: (i, 0)),
        out_shape=jax.ShapeDtypeStruct((NPAD, 64), jnp.float32),
    )(agg2, dinv_b, b2r)


def _decoder(z_pad):
    bi = 2048
    bj = 2048

    def body(a_ref, b_ref, o_ref):
        prod = lax.dot_general(a_ref[...], b_ref[...],
                               (((1,), (1,)), ((), ())),
                               preferred_element_type=jnp.float32)
        o_ref[...] = jax.nn.sigmoid(prod)

    return pl.pallas_call(
        body,
        grid=(NPAD // bi, NPAD // bj),
        in_specs=[pl.BlockSpec((bi, 64), lambda i, j: (i, 0)),
                  pl.BlockSpec((bj, 64), lambda i, j: (j, 0))],
        out_specs=pl.BlockSpec((bi, bj), lambda i, j: (i, j)),
        out_shape=jax.ShapeDtypeStruct((N, N), jnp.float32),
    )(z_pad, z_pad)


def kernel(x, edge_index, W1, b1, W2, b2):
    src = edge_index[0].astype(jnp.int32)
    dst = edge_index[1].astype(jnp.int32)
    pad = EPAD - E
    src_p = jnp.concatenate([src, jnp.full((pad,), N, jnp.int32)])
    dst_p = jnp.concatenate([dst, jnp.full((pad,), N, jnp.int32)])

    dst_deg = dst_p.reshape(N_SC, N_TILE, DEG_CHUNKS, CHUNK)
    src_t = src_p.reshape(N_TILE, AGG_CHUNKS, CHUNK)
    srcs_agg = jnp.stack([src_t, src_t + NPAD])
    dst_agg = dst_p.reshape(N_TILE, AGG_CHUNKS, CHUNK)
    ones_h = jnp.ones((CHUNK, 16), jnp.float32)
    zeros_h = jnp.zeros((ROWS_PER_TILE, 16), jnp.float32)

    x_pad = jnp.pad(x, ((0, NPAD - N), (0, 0)))
    deg_parts = _deg(dst_deg, ones_h, zeros_h)
    g1_flat, dinv_b = _mm1_scale(x_pad, W1, deg_parts)
    agg1 = _agg(g1_flat, srcs_agg, dst_agg, 128)
    g2_cat = _layer2(agg1, dinv_b, b1.reshape(1, 256), W2)
    agg2 = _agg(g2_cat.reshape(N_SC * NPAD, 32), srcs_agg, dst_agg, 32)
    z_pad = _zout(agg2, dinv_b, b2.reshape(1, 64))
    recons = _decoder(z_pad)
    return (recons, z_pad[:N])
